# Initial kernel scaffold; baseline (speedup 1.0000x reference)
import jax
import jax.numpy as jnp
from jax.experimental import pallas as pl


def kernel(dynamic, cI, cR, N, I, R, h, W1, a1_src, a1_dst, b1, W2, a2_src, a2_dst, b2, W_ih, W_hh, b_ih, b_hh, WI, bI, WR, bR, Wsir, bsir, edge_index):
    raise NotImplementedError("write your pallas kernel here")


# speedup vs baseline: 33.0031x; 33.0031x over previous
import functools

import jax
import jax.numpy as jnp
import numpy as np
from jax import lax
from jax.experimental import pallas as pl
from jax.experimental.pallas import tpu as pltpu
from jax.experimental.pallas import tpu_sc as plsc

N_NODES = 10000
N_PAD = 10112
DUMMY = N_NODES
T = 8
IN_DIM = 128
H1 = 32
HEADS = 4
D1 = HEADS * H1
DA1 = D1 + 16
H2 = 32
DA2 = H2 + 16
GRU_DIM = 64
PW = 5

N_EDGES = 320000
E_REAL = N_EDGES + N_NODES
NW = 32
EPW = 10368
E_PAD = NW * EPW
NCHUNK = 3
CE = EPW // NCHUNK
NGRP = CE // 64
ROWS_PER_TILE = N_PAD // 16


def _gat_sc_kernel(H, D):
    HD = D // H
    DA = D + 16
    mesh = plsc.VectorSubcoreMesh(
        core_axis_name="c", subcore_axis_name="s", num_cores=2, num_subcores=16)

    @functools.partial(
        pl.kernel,
        out_type=jax.ShapeDtypeStruct((2, N_PAD, DA), jnp.float32),
        mesh=mesh,
        compiler_params=pltpu.CompilerParams(
            use_tc_tiling_on_sc=False, needs_layout_passes=False),
        scratch_types=[
            pltpu.VMEM((CE,), jnp.int32),
            pltpu.VMEM((CE,), jnp.int32),
            pltpu.VMEM((64,), jnp.int32),
            pltpu.VMEM((64,), jnp.int32),
            pltpu.VMEM((64, DA), jnp.float32),
            pltpu.VMEM((64, 16), jnp.float32),
            pltpu.VMEM_SHARED((N_PAD, DA), jnp.float32),
        ],
    )
    def k(xwe_h, edt_h, srcb_h, dstb_h, zrow_h, out_h,
          src_v, dst_v, src64, dst64, gbuf, ebuf, acc):
        c = lax.axis_index("c")
        s = lax.axis_index("s")
        wid = s * 2 + c
        row0 = s * ROWS_PER_TILE
        pltpu.sync_copy(zrow_h, acc.at[pl.ds(row0, ROWS_PER_TILE)])
        plsc.subcore_barrier()

        def chunk_body(ch, cr):
            pltpu.sync_copy(srcb_h.at[wid, ch], src_v)
            pltpu.sync_copy(dstb_h.at[wid, ch], dst_v)

            def group_body(g, cr2):
                base = g * 64
                for j in range(4):
                    src64[pl.ds(j * 16, 16)] = src_v[pl.ds(base + j * 16, 16)]
                    dst64[pl.ds(j * 16, 16)] = dst_v[pl.ds(base + j * 16, 16)]
                pltpu.sync_copy(xwe_h.at[src64], gbuf)
                pltpu.sync_copy(edt_h.at[dst64], ebuf)

                def edge_body(e, cr3):
                    esv = gbuf[e, pl.ds(D, 16)]
                    edv = ebuf[e]
                    ev = esv + edv
                    ev = jnp.where(ev > 0.0, ev, 0.2 * ev)
                    ee = jnp.exp(ev)
                    gbuf[e, pl.ds(D, 16)] = ee
                    for h in range(H):
                        see = ee[h]
                        for cc in range(HD // 16):
                            sl = pl.ds(h * HD + cc * 16, 16)
                            gbuf[e, sl] = gbuf[e, sl] * see
                    return cr3
                lax.fori_loop(0, 64, edge_body, 0, unroll=8)
                pltpu.sync_copy(gbuf, acc.at[dst64], add=True)
                return cr2
            lax.fori_loop(0, NGRP, group_body, 0)
            return cr
        lax.fori_loop(0, NCHUNK, chunk_body, 0)
        plsc.subcore_barrier()
        pltpu.sync_copy(acc.at[pl.ds(row0, ROWS_PER_TILE)],
                        out_h.at[c, pl.ds(row0, ROWS_PER_TILE)])

    return k


_sc_l1 = _gat_sc_kernel(HEADS, D1)
_sc_l2 = _gat_sc_kernel(1, H2)


def _k0_body(dyn_ref, w1_ref, a1s_ref, a1d_ref, xwe_ref, edt_ref):
    x = dyn_ref[0]
    xw = jnp.dot(x, w1_ref[...], preferred_element_type=jnp.float32)
    es = jnp.dot(xw, a1s_ref[...], preferred_element_type=jnp.float32)
    xwe_ref[0] = jnp.concatenate([xw, es], axis=1)
    edt_ref[0] = jnp.dot(xw, a1d_ref[...], preferred_element_type=jnp.float32)


BK = N_PAD // 4


def _k0(dyn_pad, W1, A1s, A1d):
    return pl.pallas_call(
        _k0_body,
        grid=(T, N_PAD // BK),
        in_specs=[
            pl.BlockSpec((1, BK, IN_DIM), lambda t, b: (t, b, 0)),
            pl.BlockSpec((IN_DIM, D1), lambda t, b: (0, 0)),
            pl.BlockSpec((D1, 16), lambda t, b: (0, 0)),
            pl.BlockSpec((D1, 16), lambda t, b: (0, 0)),
        ],
        out_specs=[
            pl.BlockSpec((1, BK, DA1), lambda t, b: (t, b, 0)),
            pl.BlockSpec((1, BK, 16), lambda t, b: (t, b, 0)),
        ],
        out_shape=[
            jax.ShapeDtypeStruct((T, N_PAD, DA1), jnp.float32),
            jax.ShapeDtypeStruct((T, N_PAD, 16), jnp.float32),
        ],
    )(dyn_pad, W1, A1s, A1d)


def _elu(x):
    return jnp.where(x > 0.0, x, jnp.exp(x) - 1.0)


def _mid_body(acc_ref, s_ref, b1_ref, w2_ref, a2s_ref, a2d_ref,
              xwe2_ref, edt2_ref):
    a = acc_ref[0] + acc_ref[1]
    msg = a[:, 0:D1]
    den = a[:, D1:D1 + HEADS]
    denb = jnp.dot(den, s_ref[...], preferred_element_type=jnp.float32)
    h1 = _elu(msg / (denb + 1e-16) + b1_ref[...])
    xw2 = jnp.dot(h1, w2_ref[...], preferred_element_type=jnp.float32)
    es2 = jnp.dot(xw2, a2s_ref[...], preferred_element_type=jnp.float32)
    xwe2_ref[...] = jnp.concatenate([xw2, es2], axis=1)
    edt2_ref[...] = jnp.dot(xw2, a2d_ref[...], preferred_element_type=jnp.float32)


def _mid(acc1, S, b1, W2, A2s, A2d):
    return pl.pallas_call(
        _mid_body,
        grid=(N_PAD // BK,),
        in_specs=[
            pl.BlockSpec((2, BK, DA1), lambda b: (0, b, 0)),
            pl.BlockSpec((HEADS, D1), lambda b: (0, 0)),
            pl.BlockSpec((1, D1), lambda b: (0, 0)),
            pl.BlockSpec((D1, H2), lambda b: (0, 0)),
            pl.BlockSpec((H2, 16), lambda b: (0, 0)),
            pl.BlockSpec((H2, 16), lambda b: (0, 0)),
        ],
        out_specs=[
            pl.BlockSpec((BK, DA2), lambda b: (b, 0)),
            pl.BlockSpec((BK, 16), lambda b: (b, 0)),
        ],
        out_shape=[
            jax.ShapeDtypeStruct((N_PAD, DA2), jnp.float32),
            jax.ShapeDtypeStruct((N_PAD, 16), jnp.float32),
        ],
    )(acc1, S, b1, W2, A2s, A2d)


def _fin_body(acc_ref, b2_ref, h_ref, wih_ref, bih_ref, whh_ref, bhh_ref,
              wi64_ref, wic_ref, bi_ref, wr64_ref, wrc_ref, br_ref,
              ws64_ref, wsc_ref, bs_ref, ct_ref, it_ref, rt_ref, nv_ref,
              eye_ref,
              pi_ref, pr_ref, phi_ref, phr_ref, hout_ref):
    a = acc_ref[0] + acc_ref[1]
    den = a[:, H2:H2 + 1]
    x2 = _elu(a[:, 0:H2] / (den + 1e-16) + b2_ref[...])
    rows = lax.broadcasted_iota(jnp.int32, (N_PAD, H2), 0)
    x2 = jnp.where(rows < N_NODES, x2, 0.0)
    cur = jnp.sum(x2, axis=0, keepdims=True) * (1.0 / N_NODES)

    h = h_ref[...]
    gi = jnp.dot(cur, wih_ref[...], preferred_element_type=jnp.float32) + bih_ref[...]
    gh = jnp.dot(h, whh_ref[...], preferred_element_type=jnp.float32) + bhh_ref[...]
    i_r, i_z, i_n = gi[:, 0:64], gi[:, 64:128], gi[:, 128:192]
    h_r, h_z, h_n = gh[:, 0:64], gh[:, 64:128], gh[:, 128:192]
    r = jax.nn.sigmoid(i_r + h_r)
    z = jax.nn.sigmoid(i_z + h_z)
    nn_ = jnp.tanh(i_n + r * h_n)
    hn = (1.0 - z) * nn_ + z * h
    hout_ref[...] = hn

    ct = ct_ref[...]
    pi_ref[...] = (jnp.dot(hn, wi64_ref[...], preferred_element_type=jnp.float32)
                   + jnp.dot(ct, wic_ref[...], preferred_element_type=jnp.float32)
                   + bi_ref[...])
    pr_ref[...] = (jnp.dot(hn, wr64_ref[...], preferred_element_type=jnp.float32)
                   + jnp.dot(ct, wrc_ref[...], preferred_element_type=jnp.float32)
                   + br_ref[...])
    pres = (jnp.dot(hn, ws64_ref[...], preferred_element_type=jnp.float32)
            + jnp.dot(ct, wsc_ref[...], preferred_element_type=jnp.float32)
            + bs_ref[...])
    al = jax.nn.sigmoid(pres[:, 0:1])
    be = jax.nn.sigmoid(pres[:, 1:2])

    nv = nv_ref[...]
    last_i = it_ref[...]
    last_r = rt_ref[...]
    phi = jnp.zeros((1, PW), jnp.float32)
    phr = jnp.zeros((1, PW), jnp.float32)
    di = jnp.zeros((1, 1), jnp.float32)
    dr = jnp.zeros((1, 1), jnp.float32)
    for i in range(PW):
        if i > 0:
            last_i = last_i + di
            last_r = last_r + dr
        last_s = nv - last_i - last_r
        di = al * last_i * (last_s / nv) - be * last_i
        dr = be * last_i
        phi = phi + di * eye_ref[i:i + 1, :]
        phr = phr + dr * eye_ref[i:i + 1, :]
    phi_ref[...] = phi
    phr_ref[...] = phr


def _fin(acc2, b2, h, Wih, bih, Whh, bhh, Wi64, Wic, bi, Wr64, Wrc, br,
         Ws64, Wsc, bs, ct, it, rt, nv, eye5):
    return pl.pallas_call(
        _fin_body,
        out_shape=[
            jax.ShapeDtypeStruct((1, PW), jnp.float32),
            jax.ShapeDtypeStruct((1, PW), jnp.float32),
            jax.ShapeDtypeStruct((1, PW), jnp.float32),
            jax.ShapeDtypeStruct((1, PW), jnp.float32),
            jax.ShapeDtypeStruct((1, GRU_DIM), jnp.float32),
        ],
    )(acc2, b2, h, Wih, bih, Whh, bhh, Wi64, Wic, bi, Wr64, Wrc, br,
      Ws64, Wsc, bs, ct, it, rt, nv, eye5)


def _impl(dynamic, cI, cR, N, I, R, h, W1, a1_src, a1_dst, b1,
          W2, a2_src, a2_dst, b2, W_ih, W_hh, b_ih, b_hh,
          WI, bI, WR, bR, Wsir, bsir, edge_index):
    f32 = jnp.float32
    dyn_pad = jnp.transpose(
        jnp.pad(dynamic, ((0, N_PAD - N_NODES), (0, 0), (0, 0))), (1, 0, 2))
    loops = jnp.arange(N_NODES, dtype=jnp.int32)
    padv = jnp.full((E_PAD - E_REAL,), DUMMY, jnp.int32)
    src = jnp.concatenate([edge_index[0].astype(jnp.int32), loops, padv])
    dst = jnp.concatenate([edge_index[1].astype(jnp.int32), loops, padv])
    srcb = src.reshape(NW, NCHUNK, CE)
    dstb = dst.reshape(NW, NCHUNK, CE)

    eye4 = jnp.eye(HEADS, dtype=f32)
    zpad12 = jnp.zeros((D1, 16 - HEADS), f32)
    A1s = jnp.concatenate(
        [(a1_src[:, :, None] * eye4[:, None, :]).reshape(D1, HEADS), zpad12],
        axis=1)
    A1d = jnp.concatenate(
        [(a1_dst[:, :, None] * eye4[:, None, :]).reshape(D1, HEADS), zpad12],
        axis=1)
    zpad15 = jnp.zeros((H2, 15), f32)
    A2s = jnp.concatenate([a2_src[0][:, None], zpad15], axis=1)
    A2d = jnp.concatenate([a2_dst[0][:, None], zpad15], axis=1)
    S = (np.arange(HEADS)[:, None] == (np.arange(D1)[None, :] // H1)).astype(np.float32)
    S = jnp.asarray(S)

    zrow1 = jnp.zeros((ROWS_PER_TILE, DA1), f32)
    zrow2 = jnp.zeros((ROWS_PER_TILE, DA2), f32)

    Wih = W_ih.T
    Whh = W_hh.T
    bih = b_ih.reshape(1, -1)
    bhh = b_hh.reshape(1, -1)
    Wi64 = WI[:, 0:GRU_DIM].T
    Wic = WI[:, GRU_DIM:GRU_DIM + 2].T
    Wr64 = WR[:, 0:GRU_DIM].T
    Wrc = WR[:, GRU_DIM:GRU_DIM + 2].T
    Ws64 = Wsir[:, 0:GRU_DIM].T
    Wsc = Wsir[:, GRU_DIM:GRU_DIM + 2].T
    b1r = b1.reshape(1, -1)
    b2r = b2.reshape(1, -1)
    bir = bI.reshape(1, -1)
    brr = bR.reshape(1, -1)
    bsr = bsir.reshape(1, -1)
    nv = N.reshape(1, 1)
    eye5 = jnp.eye(PW, dtype=f32)

    XWE, EDT = _k0(dyn_pad, W1, A1s, A1d)

    pis, prs, phis, phrs = [], [], [], []
    hcur = h
    for t in range(T):
        acc1 = _sc_l1(XWE[t], EDT[t], srcb, dstb, zrow1)
        xwe2, edt2 = _mid(acc1, S, b1r, W2, A2s, A2d)
        acc2 = _sc_l2(xwe2, edt2, srcb, dstb, zrow2)
        ct = jnp.stack([cI[t], cR[t]]).reshape(1, 2)
        it = I[t].reshape(1, 1)
        rt = R[t].reshape(1, 1)
        pi, pr, phi, phr, hcur = _fin(
            acc2, b2r, hcur, Wih, bih, Whh, bhh, Wi64, Wic, bir,
            Wr64, Wrc, brr, Ws64, Wsc, bsr, ct, it, rt, nv, eye5)
        pis.append(pi)
        prs.append(pr)
        phis.append(phi)
        phrs.append(phr)

    new_I = jnp.stack(pis, axis=1)
    new_R = jnp.stack(prs, axis=1)
    phy_I = jnp.stack(phis, axis=1)
    phy_R = jnp.stack(phrs, axis=1)
    return (new_I, new_R, phy_I, phy_R, hcur)


kernel = jax.jit(_impl)
